# Initial kernel scaffold; baseline (speedup 1.0000x reference)
import jax
import jax.numpy as jnp
from jax.experimental import pallas as pl


def kernel(input_ids, logits, xi):
    raise NotImplementedError("write your pallas kernel here")


# speedup vs baseline: 7.9246x; 7.9246x over previous
import functools

import jax
import jax.numpy as jnp
from jax import lax
from jax.experimental import pallas as pl
from jax.experimental.pallas import tpu as pltpu
from jax.experimental.pallas import tpu_sc as plsc

_VOCAB = 100000
_BATCH = 64
_TOP_P = 0.9
_LANES = 16
_XI_CHUNK = 20000
_NBINS = 512

_LN2 = 0.6931471805599453
_SQRT2 = 1.4142135623730951


def _neg_log(x):
    u = plsc.bitcast(x, jnp.uint32)
    ex = (u >> jnp.uint32(23)).astype(jnp.int32) - 127
    m = plsc.bitcast(
        (u & jnp.uint32(0x007FFFFF)) | jnp.uint32(0x3F800000), jnp.float32
    )
    big = m > jnp.float32(_SQRT2)
    r = jnp.where(big, m * jnp.float32(0.5), m)
    n = (ex + big.astype(jnp.int32)).astype(jnp.float32)
    s = (r - jnp.float32(1.0)) / (r + jnp.float32(1.0))
    s2 = s * s
    p = jnp.float32(1.0 / 9.0)
    for c in (1.0 / 7.0, 1.0 / 5.0, 1.0 / 3.0, 1.0):
        p = p * s2 + jnp.float32(c)
    lnr = jnp.float32(2.0) * s * p
    return -(n * jnp.float32(_LN2) + lnr)


def _process_row(r, logits_hbm, xi_hbm, out_hbm, e_v, hist, xi_buf, tok_v):
    lane = lax.iota(jnp.int32, _LANES)
    n_vec = _VOCAB // _LANES

    pltpu.sync_copy(logits_hbm.at[pl.ds(r * _VOCAB, _VOCAB)], e_v)

    def max_body(i, mx):
        return jnp.maximum(mx, e_v[pl.ds(i * _LANES, _LANES)])

    mx = lax.fori_loop(
        0, n_vec, max_body, jnp.full((_LANES,), -jnp.inf, jnp.float32)
    )
    m = jnp.max(mx)
    m_vec = jnp.full((_LANES,), m, jnp.float32)

    def exp_body(i, z):
        sl = pl.ds(i * _LANES, _LANES)
        ev = jnp.exp(e_v[sl] - m_vec)
        e_v[sl] = ev
        return z + ev

    zv = lax.fori_loop(0, n_vec, exp_body, jnp.zeros((_LANES,), jnp.float32))
    z = jnp.sum(zv)
    budget = jnp.float32(_TOP_P) * z

    base = jnp.uint32(0)
    for lev, (sh, width) in enumerate(((23, 512), (14, 512), (5, 512), (0, 32))):

        def zero_body(i, _):
            hist[pl.ds(i * _LANES, _LANES)] = jnp.zeros((_LANES,), jnp.float32)
            return 0

        lax.fori_loop(0, (width * _LANES) // _LANES, zero_body, 0)

        base_vec = jnp.full((_LANES,), base, jnp.uint32)
        span = jnp.uint32(1 << (sh + 9)) if lev else None

        def hist_body(i, _):
            ev = e_v[pl.ds(i * _LANES, _LANES)]
            u = plsc.bitcast(ev, jnp.uint32)
            rel = u - base_vec
            if lev == 0:
                binv = (rel >> jnp.uint32(sh)).astype(jnp.int32)
                idx = (binv << 4) | lane
                plsc.addupdate_scatter(hist, [idx], ev)
            else:
                ok = rel < span
                binv = (rel >> jnp.uint32(sh)).astype(jnp.int32)
                idx = jnp.where(ok, (binv << 4) | lane, lane)
                plsc.addupdate_scatter(hist, [idx], ev, mask=ok)
            return 0

        lax.fori_loop(0, n_vec, hist_body, 0)

        def scan_body(j, carry):
            acc, bstar, found = carry
            b = width - 1 - j
            w = jnp.sum(hist[pl.ds(b * _LANES, _LANES)])
            na = acc + w
            cross = jnp.logical_and(jnp.logical_not(found), na >= budget_lvl)
            bstar = jnp.where(cross, b, bstar)
            found = jnp.logical_or(found, cross)
            acc = jnp.where(found, acc, na)
            return acc, bstar, found

        budget_lvl = budget
        acc, bstar, found = lax.fori_loop(
            0, width, scan_body, (jnp.float32(0.0), jnp.int32(0), False)
        )
        bstar = jnp.where(found, bstar, 0)
        budget = budget - acc
        base = base + (bstar.astype(jnp.uint32) << jnp.uint32(sh))

    t_vec = jnp.full((_LANES,), base, jnp.uint32)
    et_vec = plsc.bitcast(t_vec, jnp.float32)
    r_vec = jnp.full((_LANES,), budget, jnp.float32)

    def chunk_body(c, carry):
        pltpu.sync_copy(
            xi_hbm.at[pl.ds(r * _VOCAB + c * _XI_CHUNK, _XI_CHUNK)], xi_buf
        )

        def score_body(i, carry):
            best_E, best_e, best_i, cnt = carry
            ev = e_v[pl.ds(c * _XI_CHUNK + i * _LANES, _LANES)]
            u = plsc.bitcast(ev, jnp.uint32)
            E = _neg_log(xi_buf[pl.ds(i * _LANES, _LANES)])
            eq = u == t_vec
            gt = u > t_vec
            eqi = jnp.where(eq, jnp.int32(1), jnp.int32(0))
            pc = plsc.cumsum(eqi)
            rank_f = (cnt + pc - 1).astype(jnp.float32)
            inc = jnp.logical_or(gt, jnp.logical_and(eq, rank_f * et_vec < r_vec))
            cnt = cnt + plsc.all_reduce_population_count(eq)
            better = jnp.logical_and(inc, E * best_e < best_E * ev)
            gidx = jnp.full((_LANES,), c * _XI_CHUNK + i * _LANES, jnp.int32) + lane
            best_E = jnp.where(better, E, best_E)
            best_e = jnp.where(better, ev, best_e)
            best_i = jnp.where(better, gidx, best_i)
            return best_E, best_e, best_i, cnt

        return lax.fori_loop(0, _XI_CHUNK // _LANES, score_body, carry)

    init = (
        jnp.full((_LANES,), jnp.inf, jnp.float32),
        jnp.full((_LANES,), 1.0, jnp.float32),
        jnp.full((_LANES,), 0, jnp.int32),
        jnp.zeros((_LANES,), jnp.int32),
    )
    best_E, best_e, best_i, _ = lax.fori_loop(
        0, _VOCAB // _XI_CHUNK, chunk_body, init
    )
    s = best_E / best_e
    smin = jnp.min(s)
    win = jnp.min(jnp.where(s == smin, best_i, jnp.int32(2**31 - 1)))
    tok_v[...] = jnp.full((_LANES,), win, jnp.int32)
    pltpu.sync_copy(tok_v, out_hbm.at[pl.ds(r * _LANES, _LANES)])


def _sc_select(logits, xi):
    mesh = plsc.VectorSubcoreMesh(core_axis_name="c", subcore_axis_name="s")

    @functools.partial(
        pl.kernel,
        out_type=jax.ShapeDtypeStruct((_BATCH * _LANES,), jnp.int32),
        mesh=mesh,
        scratch_types=[
            pltpu.VMEM((_VOCAB,), jnp.float32),
            pltpu.VMEM((_NBINS * _LANES,), jnp.float32),
            pltpu.VMEM((_XI_CHUNK,), jnp.float32),
            pltpu.VMEM((_LANES,), jnp.int32),
        ],
        compiler_params=pltpu.CompilerParams(needs_layout_passes=False),
    )
    def run(logits_hbm, xi_hbm, out_hbm, e_v, hist, xi_buf, tok_v):
        wid = lax.axis_index("s") * 2 + lax.axis_index("c")
        for rr in range(2):
            _process_row(
                wid * 2 + rr, logits_hbm, xi_hbm, out_hbm, e_v, hist, xi_buf, tok_v
            )

    return run(logits.reshape(-1), xi.reshape(-1))


def _tc_finish_body(tok_ref, logits_ref, out_ref):
    col = lax.broadcasted_iota(jnp.int32, (1, _VOCAB), 1)
    blk = pl.program_id(0)
    for j in range(8):
        tok = tok_ref[blk * 8 + j]
        row = logits_ref[pl.ds(j, 1), :]
        out_ref[pl.ds(j, 1), :] = jnp.where(
            col == tok, row + jnp.float32(50.0), row
        )


def _tc_finish(logits, tokens):
    return pl.pallas_call(
        _tc_finish_body,
        grid=(_BATCH // 8,),
        in_specs=[
            pl.BlockSpec(memory_space=pltpu.SMEM),
            pl.BlockSpec((8, _VOCAB), lambda i: (i, 0)),
        ],
        out_specs=pl.BlockSpec((8, _VOCAB), lambda i: (i, 0)),
        out_shape=jax.ShapeDtypeStruct((_BATCH, _VOCAB), jnp.float32),
    )(tokens, logits)


def kernel(input_ids, logits, xi):
    del input_ids
    toks = _sc_select(logits, xi)
    return _tc_finish(logits, toks.reshape(_BATCH, _LANES)[:, 0])
